# baseline (speedup 1.0000x reference)
import functools

import jax
import jax.numpy as jnp
from jax import lax
from jax.experimental import pallas as pl
from jax.experimental.pallas import tpu as pltpu
from jax.experimental.pallas import tpu_sc as plsc

_NC = 2
_NS = 16
_L = 16
_R = 49920
_DH = 4
_U = 4


def kernel(text_indices, context_indices, w_in, w_out):
    B = text_indices.shape[0]
    N, E = w_in.shape
    assert E == _NC * _NS and B % (_NS * _L * _U) == 0
    tail = N - 2 * _R
    assert 0 < tail <= 256
    half = B // 2
    b_per_t = B // _NS
    mesh = plsc.VectorSubcoreMesh(core_axis_name="c", subcore_axis_name="s")

    @functools.partial(
        pl.kernel,
        mesh=mesh,
        out_type=jax.ShapeDtypeStruct((_NC, B), jnp.float32),
        compiler_params=pltpu.CompilerParams(needs_layout_passes=False),
        scratch_types=[
            pltpu.VMEM((_R,), jnp.float32),
            pltpu.VMEM((tail,), jnp.float32),
            pltpu.VMEM((B,), jnp.int32),
            pltpu.VMEM((half,), jnp.int32),
            pltpu.VMEM((B,), jnp.float32),
            pltpu.VMEM((B,), jnp.float32),
            pltpu.VMEM((_DH, B // _NS), jnp.float32),
            pltpu.VMEM_SHARED((_NS, B), jnp.float32),
        ],
    )
    def sc_kernel(ti_hbm, ci_hbm, wt_in_hbm, wt_out_hbm, out_hbm,
                  row_v, tail_v, ti_v, cic_v, xfull, pc_v, pbuf, p_sp):
        c = lax.axis_index("c")
        t = lax.axis_index("s")
        d = c * _NS + t

        pltpu.sync_copy(ti_hbm, ti_v)

        for r in range(2):
            if r == 0:
                pltpu.sync_copy(wt_in_hbm.at[d, pl.ds(0, _R)], row_v)
            else:
                pltpu.sync_copy(wt_in_hbm.at[d, pl.ds(_R, _R)], row_v)
                pltpu.sync_copy(wt_in_hbm.at[d, pl.ds(2 * _R, tail)], tail_v)

            def xbody(g, carry):
                base = g * (_U * _L)
                for u in range(_U):
                    sl = pl.ds(base + u * _L, _L)
                    iv = ti_v[sl]
                    if r == 0:
                        m = iv < _R
                        gv = plsc.load_gather(row_v, [iv], mask=m)
                        xfull[sl] = jnp.where(m, gv, 0.0)
                    else:
                        m1 = (iv >= _R) & (iv < 2 * _R)
                        g1 = plsc.load_gather(row_v, [iv - _R], mask=m1)
                        m2 = iv >= 2 * _R
                        g2 = plsc.load_gather(tail_v, [iv - 2 * _R], mask=m2)
                        xfull[sl] = (xfull[sl] + jnp.where(m1, g1, 0.0)
                                     + jnp.where(m2, g2, 0.0))
                return carry

            lax.fori_loop(0, B // (_U * _L), xbody, 0)

        for r in range(2):
            if r == 0:
                pltpu.sync_copy(wt_out_hbm.at[d, pl.ds(0, _R)], row_v)
            else:
                pltpu.sync_copy(wt_out_hbm.at[d, pl.ds(_R, _R)], row_v)
                pltpu.sync_copy(wt_out_hbm.at[d, pl.ds(2 * _R, tail)], tail_v)
            for ih in range(2):
                hbase = ih * half
                pltpu.sync_copy(ci_hbm.at[pl.ds(hbase, half)], cic_v)

                def cbody(g, carry):
                    base = g * (_U * _L)
                    for u in range(_U):
                        o = base + u * _L
                        sl = pl.ds(hbase + o, _L)
                        iv = cic_v[pl.ds(o, _L)]
                        if r == 0:
                            m = iv < _R
                            gv = plsc.load_gather(row_v, [iv], mask=m)
                            pc_v[sl] = jnp.where(m, gv, 0.0) * xfull[sl]
                        else:
                            m1 = (iv >= _R) & (iv < 2 * _R)
                            g1 = plsc.load_gather(row_v, [iv - _R], mask=m1)
                            m2 = iv >= 2 * _R
                            g2 = plsc.load_gather(
                                tail_v, [iv - 2 * _R], mask=m2)
                            cv = (jnp.where(m1, g1, 0.0)
                                  + jnp.where(m2, g2, 0.0))
                            pc_v[sl] = pc_v[sl] + cv * xfull[sl]
                    return carry

                lax.fori_loop(0, half // (_U * _L), cbody, 0)

        pltpu.sync_copy(pc_v, p_sp.at[t])
        plsc.subcore_barrier()

        bbase = t * b_per_t
        for dchunk in range(_NS // _DH):
            dbase = dchunk * _DH
            pltpu.sync_copy(p_sp.at[pl.ds(dbase, _DH), pl.ds(bbase, b_per_t)],
                            pbuf)

            def rbody(v, carry):
                sl = pl.ds(v * _L, _L)
                acc = pbuf[0, sl]
                for dd in range(1, _DH):
                    acc = acc + pbuf[dd, sl]
                if dchunk:
                    acc = acc + pc_v[sl]
                pc_v[sl] = acc
                return carry

            lax.fori_loop(0, b_per_t // _L, rbody, 0)
        pltpu.sync_copy(pc_v.at[pl.ds(0, b_per_t)],
                        out_hbm.at[c, pl.ds(bbase, b_per_t)])

    partials = sc_kernel(text_indices, context_indices, w_in.T, w_out.T)
    return partials[0] + partials[1]


# speedup vs baseline: 1.3679x; 1.3679x over previous
import functools

import jax
import jax.numpy as jnp
from jax import lax
from jax.experimental import pallas as pl
from jax.experimental.pallas import tpu as pltpu
from jax.experimental.pallas import tpu_sc as plsc

_NC = 2
_NS = 16
_L = 16
_R = 49920
_DH = 4
_U = 4


def kernel(text_indices, context_indices, w_in, w_out):
    B = text_indices.shape[0]
    N, E = w_in.shape
    assert E == _NC * _NS and B % (_NS * _L * _U) == 0
    tail = N - 2 * _R
    assert 0 < tail <= 256
    half = B // 2
    b_per_t = B // _NS
    mesh = plsc.VectorSubcoreMesh(core_axis_name="c", subcore_axis_name="s")

    @functools.partial(
        pl.kernel,
        mesh=mesh,
        out_type=jax.ShapeDtypeStruct((_NC, B), jnp.float32),
        compiler_params=pltpu.CompilerParams(needs_layout_passes=False),
        scratch_types=[
            pltpu.VMEM((_R,), jnp.float32),
            pltpu.VMEM((tail,), jnp.float32),
            pltpu.VMEM((B,), jnp.int32),
            pltpu.VMEM((half,), jnp.int32),
            pltpu.VMEM((B,), jnp.float32),
            pltpu.VMEM((B,), jnp.float32),
            pltpu.VMEM((_DH, B // _NS), jnp.float32),
            pltpu.VMEM_SHARED((_NS, B), jnp.float32),
        ],
    )
    def sc_kernel(ti_hbm, ci_hbm, wt_in_hbm, wt_out_hbm, out_hbm,
                  row_v, tail_v, ti_v, cic_v, xfull, pc_v, pbuf, p_sp):
        c = lax.axis_index("c")
        t = lax.axis_index("s")
        d = c * _NS + t

        pltpu.sync_copy(ti_hbm, ti_v)

        for r in range(2):
            if r == 0:
                pltpu.sync_copy(wt_in_hbm.at[d, pl.ds(0, _R)], row_v)
            else:
                pltpu.sync_copy(wt_in_hbm.at[d, pl.ds(_R, _R)], row_v)
                pltpu.sync_copy(wt_in_hbm.at[d, pl.ds(2 * _R, tail)], tail_v)

            @plsc.parallel_loop(0, B, step=_L, unroll=_U)
            def xbody(i):
                sl = pl.ds(i, _L)
                iv = ti_v[sl]
                if r == 0:
                    m = iv < _R
                    gv = plsc.load_gather(row_v, [iv], mask=m)
                    xfull[sl] = jnp.where(m, gv, 0.0)
                else:
                    m1 = (iv >= _R) & (iv < 2 * _R)
                    g1 = plsc.load_gather(row_v, [iv - _R], mask=m1)
                    m2 = iv >= 2 * _R
                    g2 = plsc.load_gather(tail_v, [iv - 2 * _R], mask=m2)
                    xfull[sl] = (xfull[sl] + jnp.where(m1, g1, 0.0)
                                 + jnp.where(m2, g2, 0.0))

        for r in range(2):
            if r == 0:
                pltpu.sync_copy(wt_out_hbm.at[d, pl.ds(0, _R)], row_v)
            else:
                pltpu.sync_copy(wt_out_hbm.at[d, pl.ds(_R, _R)], row_v)
                pltpu.sync_copy(wt_out_hbm.at[d, pl.ds(2 * _R, tail)], tail_v)
            for ih in range(2):
                hbase = ih * half
                pltpu.sync_copy(ci_hbm.at[pl.ds(hbase, half)], cic_v)

                @plsc.parallel_loop(0, half, step=_L, unroll=_U)
                def cbody(o):
                    sl = pl.ds(hbase + o, _L)
                    iv = cic_v[pl.ds(o, _L)]
                    if r == 0:
                        m = iv < _R
                        gv = plsc.load_gather(row_v, [iv], mask=m)
                        pc_v[sl] = jnp.where(m, gv, 0.0) * xfull[sl]
                    else:
                        m1 = (iv >= _R) & (iv < 2 * _R)
                        g1 = plsc.load_gather(row_v, [iv - _R], mask=m1)
                        m2 = iv >= 2 * _R
                        g2 = plsc.load_gather(tail_v, [iv - 2 * _R], mask=m2)
                        cv = (jnp.where(m1, g1, 0.0)
                              + jnp.where(m2, g2, 0.0))
                        pc_v[sl] = pc_v[sl] + cv * xfull[sl]

        pltpu.sync_copy(pc_v, p_sp.at[t])
        plsc.subcore_barrier()

        bbase = t * b_per_t
        for dchunk in range(_NS // _DH):
            dbase = dchunk * _DH
            pltpu.sync_copy(p_sp.at[pl.ds(dbase, _DH), pl.ds(bbase, b_per_t)],
                            pbuf)

            @plsc.parallel_loop(0, b_per_t, step=_L, unroll=_U)
            def rbody(v):
                sl = pl.ds(v, _L)
                acc = pbuf[0, sl]
                for dd in range(1, _DH):
                    acc = acc + pbuf[dd, sl]
                if dchunk:
                    acc = acc + pc_v[sl]
                pc_v[sl] = acc
        pltpu.sync_copy(pc_v.at[pl.ds(0, b_per_t)],
                        out_hbm.at[c, pl.ds(bbase, b_per_t)])

    partials = sc_kernel(text_indices, context_indices, w_in.T, w_out.T)
    return partials[0] + partials[1]


# speedup vs baseline: 1.3973x; 1.0215x over previous
import functools

import jax
import jax.numpy as jnp
from jax import lax
from jax.experimental import pallas as pl
from jax.experimental.pallas import tpu as pltpu
from jax.experimental.pallas import tpu_sc as plsc

_NC = 2
_NS = 16
_L = 16
_R = 49920
_DH = 4
_U = 4


def kernel(text_indices, context_indices, w_in, w_out):
    B = text_indices.shape[0]
    N, E = w_in.shape
    assert E == _NC * _NS and B % (_NS * _L * _U) == 0
    tail = N - 2 * _R
    assert 0 < tail <= 256
    half = B // 2
    b_per_t = B // _NS
    mesh = plsc.VectorSubcoreMesh(core_axis_name="c", subcore_axis_name="s")

    @functools.partial(
        pl.kernel,
        mesh=mesh,
        out_type=jax.ShapeDtypeStruct((_NC, B), jnp.float32),
        compiler_params=pltpu.CompilerParams(needs_layout_passes=False),
        scratch_types=[
            pltpu.VMEM((_R + tail,), jnp.float32),
            pltpu.VMEM((tail,), jnp.float32),
            pltpu.VMEM((B,), jnp.int32),
            pltpu.VMEM((half,), jnp.int32),
            pltpu.VMEM((B,), jnp.float32),
            pltpu.VMEM((B,), jnp.float32),
            pltpu.VMEM((_DH, B // _NS), jnp.float32),
            pltpu.VMEM_SHARED((_NS, B), jnp.float32),
        ],
    )
    def sc_kernel(ti_hbm, ci_hbm, wt_in_hbm, wt_out_hbm, out_hbm,
                  row_v, tail_v, ti_v, cic_v, xfull, pc_v, pbuf, p_sp):
        c = lax.axis_index("c")
        t = lax.axis_index("s")
        d = c * _NS + t

        pltpu.sync_copy(ti_hbm, ti_v)

        for r in range(2):
            if r == 0:
                pltpu.sync_copy(wt_in_hbm.at[d, pl.ds(0, _R)],
                                row_v.at[pl.ds(0, _R)])
            else:
                pltpu.sync_copy(wt_in_hbm.at[d, pl.ds(_R, _R)],
                                row_v.at[pl.ds(0, _R)])
                pltpu.sync_copy(wt_in_hbm.at[d, pl.ds(2 * _R, tail)], tail_v)
                for k in range(tail // _L):
                    row_v[pl.ds(_R + k * _L, _L)] = tail_v[pl.ds(k * _L, _L)]

            @plsc.parallel_loop(0, B, step=_L, unroll=_U)
            def xbody(i):
                sl = pl.ds(i, _L)
                iv = ti_v[sl]
                if r == 0:
                    m = iv < _R
                    gv = plsc.load_gather(row_v, [iv], mask=m)
                    xfull[sl] = jnp.where(m, gv, 0.0)
                else:
                    m = iv >= _R
                    gv = plsc.load_gather(row_v, [iv - _R], mask=m)
                    xfull[sl] = xfull[sl] + jnp.where(m, gv, 0.0)

        for r in range(2):
            if r == 0:
                pltpu.sync_copy(wt_out_hbm.at[d, pl.ds(0, _R)],
                                row_v.at[pl.ds(0, _R)])
            else:
                pltpu.sync_copy(wt_out_hbm.at[d, pl.ds(_R, _R)],
                                row_v.at[pl.ds(0, _R)])
                pltpu.sync_copy(wt_out_hbm.at[d, pl.ds(2 * _R, tail)], tail_v)
                for k in range(tail // _L):
                    row_v[pl.ds(_R + k * _L, _L)] = tail_v[pl.ds(k * _L, _L)]
            for ih in range(2):
                hbase = ih * half
                pltpu.sync_copy(ci_hbm.at[pl.ds(hbase, half)], cic_v)

                @plsc.parallel_loop(0, half, step=_L, unroll=_U)
                def cbody(o):
                    sl = pl.ds(hbase + o, _L)
                    iv = cic_v[pl.ds(o, _L)]
                    if r == 0:
                        m = iv < _R
                        gv = plsc.load_gather(row_v, [iv], mask=m)
                        pc_v[sl] = jnp.where(m, gv, 0.0) * xfull[sl]
                    else:
                        m = iv >= _R
                        gv = plsc.load_gather(row_v, [iv - _R], mask=m)
                        pc_v[sl] = (pc_v[sl]
                                    + jnp.where(m, gv, 0.0) * xfull[sl])

        pltpu.sync_copy(pc_v, p_sp.at[t])
        plsc.subcore_barrier()

        bbase = t * b_per_t
        for dchunk in range(_NS // _DH):
            dbase = dchunk * _DH
            pltpu.sync_copy(p_sp.at[pl.ds(dbase, _DH), pl.ds(bbase, b_per_t)],
                            pbuf)

            @plsc.parallel_loop(0, b_per_t, step=_L, unroll=_U)
            def rbody(v):
                sl = pl.ds(v, _L)
                acc = pbuf[0, sl]
                for dd in range(1, _DH):
                    acc = acc + pbuf[dd, sl]
                if dchunk:
                    acc = acc + pc_v[sl]
                pc_v[sl] = acc
        pltpu.sync_copy(pc_v.at[pl.ds(0, b_per_t)],
                        out_hbm.at[c, pl.ds(bbase, b_per_t)])

    partials = sc_kernel(text_indices, context_indices, w_in.T, w_out.T)
    return partials[0] + partials[1]
